# Initial kernel scaffold; baseline (speedup 1.0000x reference)
import jax
import jax.numpy as jnp
from jax.experimental import pallas as pl


def kernel(x, edge_index, PE, embed_table, code_token, W, b):
    raise NotImplementedError("write your pallas kernel here")


# speedup vs baseline: 47.0027x; 47.0027x over previous
import functools

import jax
import jax.numpy as jnp
from jax import lax
from jax.experimental import pallas as pl
from jax.experimental.pallas import tpu as pltpu
from jax.experimental.pallas import tpu_sc as plsc

NC = 2
NS = 16
NW = NC * NS
LN = 16


def _round_up(a, m):
    return (a + m - 1) // m * m


def _sc_gather_hist(idx3, dst3, table, TP, D):
    RW = idx3.shape[1]
    CPW = dst3.shape[1]
    TR = TP // 128

    def body(idx_hbm, dst_hbm, tab_hbm, enc_out, deg_out,
             idx_v, rows_v, dst_v, deg_v, sem):
        cid = lax.axis_index("c")
        sid = lax.axis_index("s")
        wid = cid * NS + sid

        pltpu.sync_copy(idx_hbm.at[wid], idx_v)
        for j in range(RW):
            pltpu.async_copy(tab_hbm.at[idx_v.at[j]], rows_v, sem).wait()
            pltpu.sync_copy(rows_v, enc_out.at[pl.ds((wid * RW + j) * 128, 128)])

        pltpu.sync_copy(dst_hbm.at[wid], dst_v)
        z16 = jnp.zeros((LN,), jnp.float32)
        o16 = jnp.ones((LN,), jnp.float32)

        def zero_body(i, c):
            for cc in range(8):
                deg_v[i, pl.ds(cc * LN, LN)] = z16
            return c
        lax.fori_loop(0, TR, zero_body, 0)

        def hist_body(i, c):
            for cc in range(8):
                dvec = dst_v[i, pl.ds(cc * LN, LN)]
                plsc.addupdate_scatter(
                    deg_v, [lax.shift_right_logical(dvec, 7),
                            lax.bitwise_and(dvec, 127)], o16)
            return c
        lax.fori_loop(0, CPW, hist_body, 0)
        pltpu.sync_copy(deg_v, deg_out.at[wid])

    mesh = plsc.VectorSubcoreMesh(core_axis_name="c", subcore_axis_name="s",
                                  num_cores=NC, num_subcores=NS)
    f = pl.kernel(
        body,
        out_type=[
            jax.ShapeDtypeStruct((NW * RW * 128, D), jnp.float32),
            jax.ShapeDtypeStruct((NW, TR, 128), jnp.float32),
        ],
        mesh=mesh,
        scratch_types=[
            pltpu.VMEM((RW, 128), jnp.int32),
            pltpu.VMEM((128, D), jnp.float32),
            pltpu.VMEM((CPW, 128), jnp.int32),
            pltpu.VMEM((TR, 128), jnp.float32),
            pltpu.SemaphoreType.DMA,
        ],
        compiler_params=pltpu.CompilerParams(needs_layout_passes=False),
    )
    return f(idx3, dst3, table)


def _sc_edge_agg(src3, dst3, y128, TP):
    CPW = src3.shape[1]
    RPT = TP // NS

    chunks = []
    off = 0
    while off < RPT:
        sz = min(128, RPT - off)
        chunks.append((off, sz))
        off += sz

    def body(src_hbm, dst_hbm, y_hbm, parts_out,
             src_v, dst_v, rows_v, agg_sh, sem):
        cid = lax.axis_index("c")
        sid = lax.axis_index("s")
        wid = cid * NS + sid

        z16 = jnp.zeros((LN,), jnp.float32)

        def zero_body(i, c):
            for cc in range(8):
                rows_v[i, pl.ds(cc * LN, LN)] = z16
            return c
        lax.fori_loop(0, 128, zero_body, 0)
        for off, sz in chunks:
            pltpu.sync_copy(rows_v.at[pl.ds(0, sz)],
                            agg_sh.at[pl.ds(sid * RPT + off, sz)])
        plsc.subcore_barrier()

        def edge_body(j, c):
            pltpu.sync_copy(src_hbm.at[wid, j], src_v)
            pltpu.sync_copy(dst_hbm.at[wid, j], dst_v)
            pltpu.async_copy(y_hbm.at[src_v], rows_v, sem).wait()
            pltpu.sync_copy(rows_v, agg_sh.at[dst_v], add=True)
            return c
        lax.fori_loop(0, CPW, edge_body, 0)
        plsc.subcore_barrier()

        for off, sz in chunks:
            pltpu.sync_copy(agg_sh.at[pl.ds(sid * RPT + off, sz)],
                            rows_v.at[pl.ds(0, sz)])
            pltpu.sync_copy(rows_v.at[pl.ds(0, sz)],
                            parts_out.at[pl.ds(cid * TP + sid * RPT + off, sz)])

    mesh = plsc.VectorSubcoreMesh(core_axis_name="c", subcore_axis_name="s",
                                  num_cores=NC, num_subcores=NS)
    f = pl.kernel(
        body,
        out_type=jax.ShapeDtypeStruct((NC * TP, 128), jnp.float32),
        mesh=mesh,
        scratch_types=[
            pltpu.VMEM((128,), jnp.int32),
            pltpu.VMEM((128,), jnp.int32),
            pltpu.VMEM((128, 128), jnp.float32),
            pltpu.VMEM_SHARED((TP, 128), jnp.float32),
            pltpu.SemaphoreType.DMA,
        ],
    )
    return f(src3, dst3, y128)


def _tc_matmul(enc, W):
    NR = enc.shape[0]
    RB = 4096

    def body(e_ref, w_ref, o_ref):
        o_ref[...] = jnp.dot(e_ref[...], w_ref[...],
                             preferred_element_type=jnp.float32)

    return pl.pallas_call(
        body,
        grid=(NR // RB,),
        in_specs=[
            pl.BlockSpec((RB, enc.shape[1]), lambda i: (i, 0)),
            pl.BlockSpec(W.shape, lambda i: (0, 0)),
        ],
        out_specs=pl.BlockSpec((RB, W.shape[1]), lambda i: (i, 0)),
        out_shape=jax.ShapeDtypeStruct((NR, W.shape[1]), jnp.float32),
    )(enc, W)


def _tc_attention(logits3, enc3, PE):
    N, K, T = logits3.shape
    D = enc3.shape[2]

    def body(l_ref, e_ref, p_ref, att_ref, o_ref):
        v = l_ref[...]
        m = jnp.max(v, axis=-1, keepdims=True)
        ex = jnp.exp(v - m)
        att = ex / jnp.sum(ex, axis=-1, keepdims=True)
        att_ref[...] = att
        e = e_ref[...] + p_ref[...]
        o_ref[...] = jnp.dot(att, e, preferred_element_type=jnp.float32)

    return pl.pallas_call(
        body,
        grid=(N,),
        in_specs=[
            pl.BlockSpec((None, K, T), lambda n: (n, 0, 0)),
            pl.BlockSpec((None, T, D), lambda n: (n, 0, 0)),
            pl.BlockSpec((None, T, D), lambda n: (n, 0, 0)),
        ],
        out_specs=[
            pl.BlockSpec((None, K, T), lambda n: (n, 0, 0)),
            pl.BlockSpec((None, K, D), lambda n: (n, 0, 0)),
        ],
        out_shape=[
            jax.ShapeDtypeStruct((N, K, T), jnp.float32),
            jax.ShapeDtypeStruct((N, K, D), jnp.float32),
        ],
    )(logits3, enc3, PE)


def kernel(x, edge_index, PE, embed_table, code_token, W, b):
    T, N = x.shape
    D = embed_table.shape[1]
    OUT = W.shape[1]
    E = edge_index.shape[1]
    F = N * OUT
    USE_SC1 = True
    USE_SC2 = True
    USE_TC_ATT = True

    TP = _round_up(T + 1, 128)
    RW = -(-(T * N) // (NW * 128))
    NRI = NW * RW * 128
    CPW = -(-E // (NW * 128))
    EP = NW * CPW * 128

    src = edge_index[0]
    dst = edge_index[1]

    idx3 = jnp.concatenate(
        [x.T.reshape(-1),
         jnp.zeros((NRI - T * N,), jnp.int32)]).reshape(NW, RW, 128)
    pad_e = jnp.full((EP - E,), TP - 1, jnp.int32)
    src3 = jnp.concatenate([src, pad_e]).reshape(NW, CPW, 128)
    dst3 = jnp.concatenate([dst, pad_e]).reshape(NW, CPW, 128)

    if USE_SC1:
        enc, deg3 = _sc_gather_hist(idx3, dst3, embed_table, TP, D)
        deg = deg3.sum(axis=0).reshape(TP)[:T] + 1.0
    else:
        enc = jnp.take(embed_table, idx3.reshape(-1), axis=0)
        deg = jax.ops.segment_sum(jnp.ones((E,), jnp.float32), dst,
                                  num_segments=T) + 1.0

    xw = _tc_matmul(enc, W)

    dinv = lax.rsqrt(deg)
    y2d = (xw[:T * N].reshape(N, T, OUT).transpose(1, 0, 2).reshape(T, F)
           * dinv[:, None])

    if USE_SC2:
        y128 = jnp.pad(y2d, ((0, TP - T), (0, 128 - F)))
        parts = _sc_edge_agg(src3, dst3, y128, TP)
        agg0 = (parts[:TP] + parts[TP:])[:T, :F]
    else:
        agg0 = jax.ops.segment_sum(y2d[src], dst, num_segments=T)

    logits3 = (dinv[:, None] * (agg0 + y2d)
               + jnp.tile(b, N)[None, :]).T.reshape(N, OUT, T)
    enc3 = enc[:T * N].reshape(N, T, D)
    if USE_TC_ATT:
        att3, learned = _tc_attention(logits3, enc3, PE)
    else:
        att3 = jax.nn.softmax(logits3, axis=-1)
        learned = jnp.einsum('nkt,ntd->nkd', att3, enc3 + PE)

    code_tokens = jnp.broadcast_to(code_token[None, :, :], (N, 1, D))
    out = jnp.concatenate([code_tokens, learned], axis=1)
    return (out, att3)
